# baseline (speedup 1.0000x reference)
import functools

import jax
import jax.numpy as jnp
from jax import lax
from jax.experimental import pallas as pl
from jax.experimental.pallas import tpu as pltpu

B, K, N, E, D = 32, 32, 2048, 16384, 768
L_GNN, L_TR, H = 2, 4, 12
FF, VOCAB, N_ROLES, MAX_POS, N_HOPS = 3072, 25, 2, 64, 16

T = 256
NTILE = N // T
C = 256
S = C + 1
TOT = E + NTILE * C
NCH = TOT // C
LEN = ((NCH + TOT + 1023) // 1024) * 1024
DH = D // H
_SCALE = float(1.0 / (DH ** 0.5))

_CP = getattr(pltpu, "CompilerParams", None) or getattr(pltpu, "TPUCompilerParams")


def _bucket_edges(src, dst):
    t_e = dst // T
    order = jnp.argsort(t_e, stable=True)
    tids = jnp.arange(NTILE, dtype=dst.dtype)
    counts = jnp.sum(t_e[None, :] == tids[:, None], axis=1)
    padded = ((counts + C - 1) // C) * C
    zero = jnp.zeros((1,), counts.dtype)
    pad_off = jnp.concatenate([zero, jnp.cumsum(padded)])
    start = jnp.concatenate([zero, jnp.cumsum(counts)])
    p = jnp.arange(TOT)
    tile_p = jnp.clip(jnp.searchsorted(pad_off, p, side="right") - 1, 0, NTILE - 1)
    kk = p - pad_off[tile_p]
    valid = kk < counts[tile_p]
    eidx = order[jnp.clip(start[tile_p] + kk, 0, E - 1)]
    src_pad = jnp.where(valid, src[eidx], 0).astype(jnp.int32)
    dst_pad = jnp.where(valid, dst[eidx], -1).astype(jnp.int32)
    c = jnp.arange(NCH)
    tile_c = jnp.clip(jnp.searchsorted(pad_off, c * C, side="right") - 1, 0, NTILE - 1)
    tid_off = (tile_c * T).astype(jnp.int32)
    dst_local = dst_pad.reshape(NCH, C) - tid_off[:, None]
    bundle = jnp.concatenate(
        [tid_off, src_pad * 8, jnp.zeros((LEN - NCH - TOT,), jnp.int32)])
    return bundle, dst_local


def _gnn_kernel(xrow, batchrow, dst4, idxb, amino, Wref, bref, out,
                h_slab, h_dense, agg, tile, smem, sem):
    prot = pl.program_id(1)
    cp = pltpu.make_async_copy(idxb.at[0, 0], smem, sem)
    cp.start()
    xv = xrow[0]
    iota_v = lax.broadcasted_iota(jnp.int32, (128, N), 0)
    G = (iota_v == xv).astype(jnp.float32)
    h_dense[...] = lax.dot_general(
        G, amino[...], (((0,), (0,)), ((), ())),
        preferred_element_type=jnp.float32)
    cp.wait()

    for l in range(L_GNN):
        for j in range(D // 128):
            h_slab[j:N * 8:8, :] = h_dense[:, j * 128:(j + 1) * 128]
        agg[...] = jnp.zeros((N, D), jnp.float32)

        def chunk_body(c, _):
            base = NCH + c * C
            for mi in range(C):
                s8 = pl.multiple_of(smem[base + mi], 8)
                slab = h_slab[pl.ds(s8, 8), :]
                tile[mi:mi + 8 * S:S, :] = slab
            dstv = dst4[0, c, 0]
            iota_t = lax.broadcasted_iota(jnp.int32, (T, C), 0)
            L = (dstv[None, :] == iota_t).astype(jnp.float32)
            Hs = jnp.concatenate(
                [tile[pl.ds(j * S, C), :] for j in range(D // 128)], axis=1)
            upd = jnp.dot(L, Hs, preferred_element_type=jnp.float32)
            toff = pl.multiple_of(smem[c], T)
            agg[pl.ds(toff, T), :] = agg[pl.ds(toff, T), :] + upd
            return 0

        lax.fori_loop(0, NCH, chunk_body, 0)
        z = h_dense[...] + agg[...]
        h_dense[...] = jnp.maximum(
            jnp.dot(z, Wref[l], preferred_element_type=jnp.float32)
            + bref[l], 0.0)

    bv = batchrow[0]
    iota_b = lax.broadcasted_iota(jnp.int32, (B, N), 0)
    Lb = (bv == iota_b).astype(jnp.float32)
    pool = jnp.dot(Lb, h_dense[...], preferred_element_type=jnp.float32)
    cnt = jnp.sum(Lb, axis=1, keepdims=True)
    mean = pool / jnp.maximum(cnt, 1.0)

    @pl.when(prot == 0)
    def _():
        out[0] = mean

    @pl.when(prot == 1)
    def _():
        out[0] = out[0] + mean


def _ln(x, s, b):
    m = jnp.mean(x, -1, keepdims=True)
    v = jnp.mean(jnp.square(x - m), -1, keepdims=True)
    return (x - m) * lax.rsqrt(v + 1e-12) * s + b


def _tr_kernel(raw2, pooled2, roler, posr, hopr, table, embs, embb,
               Wq, bq, Wk, bk, Wv, bv_, Wo, bo, ln1s, ln1b,
               W1, b1, W2, b2, ln2s, ln2b, out):
    l = pl.program_id(1)
    BK = B * K // 2

    @pl.when(l == 0)
    def _():
        iota_e = lax.broadcasted_iota(jnp.int32, (128, BK), 0)
        hot = ((iota_e == roler[0]).astype(jnp.float32)
               + (iota_e == posr[0]).astype(jnp.float32)
               + (iota_e == hopr[0]).astype(jnp.float32))
        emb = lax.dot_general(
            hot, table[...], (((0,), (0,)), ((), ())),
            preferred_element_type=jnp.float32)
        h0 = raw2[...] + pooled2[...] + emb
        out[...] = _ln(h0, embs[...], embb[...])

    h = out[...]
    hb = h.astype(jnp.bfloat16)
    q = jnp.dot(hb, Wq[0], preferred_element_type=jnp.float32) + bq[0]
    kk = jnp.dot(hb, Wk[0], preferred_element_type=jnp.float32) + bk[0]
    v = jnp.dot(hb, Wv[0], preferred_element_type=jnp.float32) + bv_[0]
    nb = BK // K
    q3 = q.reshape(nb, K, D)
    k3 = kk.reshape(nb, K, D)
    v3 = v.reshape(nb, K, D)
    ctxs = []
    for hh in range(H):
        sl = slice(hh * DH, (hh + 1) * DH)
        qh, kh, vh = q3[:, :, sl], k3[:, :, sl], v3[:, :, sl]
        sc = lax.dot_general(
            qh, kh, (((2,), (2,)), ((0,), (0,))),
            preferred_element_type=jnp.float32) * _SCALE
        sc = sc - jnp.max(sc, axis=-1, keepdims=True)
        pp = jnp.exp(sc)
        pp = pp / jnp.sum(pp, axis=-1, keepdims=True)
        ctxs.append(lax.dot_general(
            pp, vh, (((2,), (1,)), ((0,), (0,))),
            preferred_element_type=jnp.float32))
    ctx = jnp.concatenate(ctxs, axis=2).reshape(BK, D)
    proj = jnp.dot(ctx.astype(jnp.bfloat16), Wo[0],
                   preferred_element_type=jnp.float32) + bo[0]
    h = _ln(h + proj, ln1s[0], ln1b[0])
    f = jax.nn.gelu(jnp.dot(h.astype(jnp.bfloat16), W1[0],
                            preferred_element_type=jnp.float32) + b1[0])
    f = jnp.dot(f.astype(jnp.bfloat16), W2[0],
                preferred_element_type=jnp.float32) + b2[0]
    out[...] = _ln(h + f, ln2s[0], ln2b[0])


def kernel(raw_features, x0, edge0, batch0, x1, edge1, batch1, role_ids,
           position_ids, hop_ids, amino_embed, gnn_W, gnn_b, role_emb,
           pos_emb, hop_emb, emb_ln_scale, emb_ln_bias, Wq, bq, Wk, bk,
           Wv, bv, Wo, bo, ln1_s, ln1_b, W1, b1, W2, b2, ln2_s, ln2_b):
    f32 = jnp.float32
    xs = jnp.concatenate([x0, x1]).astype(jnp.int32).reshape(2 * K, 1, N)
    bs = jnp.concatenate([batch0, batch1]).astype(jnp.int32).reshape(2 * K, 1, N)
    edges = jnp.concatenate([edge0, edge1]).astype(jnp.int32)
    bundle, dst_local = jax.vmap(_bucket_edges)(edges[:, 0], edges[:, 1])
    dst4 = dst_local.reshape(2 * K, NCH, 1, C)
    amino_p = jnp.zeros((128, D), f32).at[:VOCAB].set(amino_embed.astype(f32))
    b_pad = jnp.zeros((8, D), f32).at[:L_GNN].set(gnn_b.astype(f32))

    inst = lambda k, p: p * K + k
    gnn_out = pl.pallas_call(
        _gnn_kernel,
        grid=(K, 2),
        in_specs=[
            pl.BlockSpec((1, 1, N), lambda k, p: (inst(k, p), 0, 0)),
            pl.BlockSpec((1, 1, N), lambda k, p: (inst(k, p), 0, 0)),
            pl.BlockSpec((1, NCH, 1, C), lambda k, p: (inst(k, p), 0, 0, 0)),
            pl.BlockSpec((1, 1, LEN), lambda k, p: (inst(k, p), 0, 0)),
            pl.BlockSpec((128, D), lambda k, p: (0, 0)),
            pl.BlockSpec((L_GNN, D, D), lambda k, p: (0, 0, 0)),
            pl.BlockSpec((8, D), lambda k, p: (0, 0)),
        ],
        out_specs=pl.BlockSpec((1, B, D), lambda k, p: (k, 0, 0)),
        out_shape=jax.ShapeDtypeStruct((K, B, D), f32),
        scratch_shapes=[
            pltpu.VMEM((N * 8, 128), f32),
            pltpu.VMEM((N, D), f32),
            pltpu.VMEM((N, D), f32),
            pltpu.VMEM((C + 7 * S + 1, 128), f32),
            pltpu.SMEM((LEN,), jnp.int32),
            pltpu.SemaphoreType.DMA,
        ],
        compiler_params=_CP(
            dimension_semantics=("parallel", "arbitrary"),
            vmem_limit_bytes=58 * 1024 * 1024,
        ),
    )(xs, bs, dst4, bundle.reshape(2 * K, 1, LEN), amino_p, gnn_W.astype(f32), b_pad)

    pooled2 = gnn_out.transpose(1, 0, 2).reshape(B * K, D)
    raw2 = raw_features.astype(f32).reshape(B * K, D)
    roler = role_ids.astype(jnp.int32).reshape(1, B * K)
    posr = (position_ids + N_ROLES).astype(jnp.int32).reshape(1, B * K)
    hopr = (hop_ids + N_ROLES + MAX_POS).astype(jnp.int32).reshape(1, B * K)
    table = jnp.zeros((128, D), f32)
    table = table.at[:N_ROLES].set(role_emb.astype(f32))
    table = table.at[N_ROLES:N_ROLES + MAX_POS].set(pos_emb.astype(f32))
    table = table.at[N_ROLES + MAX_POS:N_ROLES + MAX_POS + N_HOPS].set(
        hop_emb.astype(f32))
    bf = jnp.bfloat16
    row = lambda a: a.astype(f32).reshape(1, -1)

    half = pl.BlockSpec((B * K // 2, D), lambda i, l: (i, 0))
    idspec = pl.BlockSpec((1, B * K // 2), lambda i, l: (0, i))
    const2 = lambda shape: pl.BlockSpec(shape, lambda i, l: (0, 0))
    lay2 = lambda d1: pl.BlockSpec((1, 1, d1), lambda i, l: (l, 0, 0))
    lay3 = lambda d1, d2: pl.BlockSpec((1, d1, d2), lambda i, l: (l, 0, 0))

    h_out = pl.pallas_call(
        _tr_kernel,
        grid=(2, L_TR),
        in_specs=[
            half, half, idspec, idspec, idspec,
            const2((128, D)), const2((1, D)), const2((1, D)),
            lay3(D, D), lay2(D), lay3(D, D), lay2(D),
            lay3(D, D), lay2(D), lay3(D, D), lay2(D),
            lay2(D), lay2(D),
            lay3(D, FF), lay2(FF), lay3(FF, D), lay2(D),
            lay2(D), lay2(D),
        ],
        out_specs=half,
        out_shape=jax.ShapeDtypeStruct((B * K, D), f32),
        compiler_params=_CP(
            dimension_semantics=("parallel", "arbitrary"),
            vmem_limit_bytes=58 * 1024 * 1024,
        ),
    )(raw2, pooled2, roler, posr, hopr, table, row(emb_ln_scale),
      row(emb_ln_bias),
      Wq.astype(bf), bq.astype(f32).reshape(L_TR, 1, D),
      Wk.astype(bf), bk.astype(f32).reshape(L_TR, 1, D),
      Wv.astype(bf), bv.astype(f32).reshape(L_TR, 1, D),
      Wo.astype(bf), bo.astype(f32).reshape(L_TR, 1, D),
      ln1_s.astype(f32).reshape(L_TR, 1, D), ln1_b.astype(f32).reshape(L_TR, 1, D),
      W1.astype(bf), b1.astype(f32).reshape(L_TR, 1, FF), W2.astype(bf), b2.astype(f32).reshape(L_TR, 1, D),
      ln2_s.astype(f32).reshape(L_TR, 1, D), ln2_b.astype(f32).reshape(L_TR, 1, D))
    return h_out.reshape(B, K, D)


# speedup vs baseline: 8.7135x; 8.7135x over previous
import functools

import jax
import jax.numpy as jnp
from jax import lax
from jax.experimental import pallas as pl
from jax.experimental.pallas import tpu as pltpu

B, K, N, E, D = 32, 32, 2048, 16384, 768
L_GNN, L_TR, H = 2, 4, 12
FF, VOCAB, N_ROLES, MAX_POS, N_HOPS = 3072, 25, 2, 64, 16

T = 256
NTILE = N // T
C = 256
S = C + 1
TOT = E + NTILE * C
NCH = TOT // C
LEN = ((NCH + TOT + 1023) // 1024) * 1024
DH = D // H
_SCALE = float(1.0 / (DH ** 0.5))

_CP = getattr(pltpu, "CompilerParams", None) or getattr(pltpu, "TPUCompilerParams")


def _bucket_edges(src, dst):
    te = dst // T
    ste, ssrc, sdst = lax.sort((te, src, dst), num_keys=1, is_stable=True)
    tids = jnp.arange(NTILE, dtype=ste.dtype)
    counts = jnp.sum(ste[None, :] == tids[:, None], axis=1)
    padded = ((counts + C - 1) // C) * C
    cpad = jnp.cumsum(padded)
    zero = jnp.zeros((1,), counts.dtype)
    pad_off = jnp.concatenate([zero, cpad[:-1]])
    start = jnp.concatenate([zero, jnp.cumsum(counts)[:-1]])
    shift = pad_off - start
    ztail = jnp.zeros((TOT - E,), jnp.int32)
    srcp = jnp.zeros((TOT,), jnp.int32)
    dstp = jnp.zeros((TOT,), jnp.int32)
    vld = jnp.zeros((TOT,), jnp.int32)
    for t in range(NTILE):
        m = ste == t
        sv = jnp.concatenate([jnp.where(m, ssrc, 0), ztail])
        dv = jnp.concatenate([jnp.where(m, sdst, 0), ztail])
        mv = jnp.concatenate([m.astype(jnp.int32), ztail])
        srcp = srcp + jnp.roll(sv, shift[t])
        dstp = dstp + jnp.roll(dv, shift[t])
        vld = vld + jnp.roll(mv, shift[t])
    dstp = jnp.where(vld > 0, dstp, -1)
    c = jnp.arange(NCH)
    tile_c = jnp.clip(jnp.searchsorted(cpad, c * C, side="right"), 0, NTILE - 1)
    tid_off = (tile_c * T).astype(jnp.int32)
    dst_local = dstp.reshape(NCH, C) - tid_off[:, None]
    bundle = jnp.concatenate(
        [tid_off, srcp * 8, jnp.zeros((LEN - NCH - TOT,), jnp.int32)])
    return bundle, dst_local


def _gnn_kernel(xrow, batchrow, dst4, idxb, amino, Wref, bref, out,
                h_slab, h_dense, agg, tile, smem, sem):
    prot = pl.program_id(1)
    cp = pltpu.make_async_copy(idxb.at[0, 0], smem, sem)
    cp.start()
    xv = xrow[0]
    iota_v = lax.broadcasted_iota(jnp.int32, (128, N), 0)
    G = (iota_v == xv).astype(jnp.float32)
    h_dense[...] = lax.dot_general(
        G, amino[...], (((0,), (0,)), ((), ())),
        preferred_element_type=jnp.float32)
    cp.wait()

    for l in range(L_GNN):
        for j in range(D // 128):
            h_slab[j:N * 8:8, :] = h_dense[:, j * 128:(j + 1) * 128]
        agg[...] = jnp.zeros((N, D), jnp.float32)

        def chunk_body(c, _):
            base = NCH + c * C
            for mi in range(C):
                s8 = pl.multiple_of(smem[base + mi], 8)
                slab = h_slab[pl.ds(s8, 8), :]
                tile[mi:mi + 8 * S:S, :] = slab
            dstv = dst4[0, c, 0]
            iota_t = lax.broadcasted_iota(jnp.int32, (T, C), 0)
            L = (dstv[None, :] == iota_t).astype(jnp.float32)
            Hs = jnp.concatenate(
                [tile[pl.ds(j * S, C), :] for j in range(D // 128)], axis=1)
            upd = jnp.dot(L, Hs, preferred_element_type=jnp.float32)
            toff = pl.multiple_of(smem[c], T)
            agg[pl.ds(toff, T), :] = agg[pl.ds(toff, T), :] + upd
            return 0

        lax.fori_loop(0, NCH, chunk_body, 0)
        z = h_dense[...] + agg[...]
        h_dense[...] = jnp.maximum(
            jnp.dot(z, Wref[l], preferred_element_type=jnp.float32)
            + bref[l], 0.0)

    bv = batchrow[0]
    iota_b = lax.broadcasted_iota(jnp.int32, (B, N), 0)
    Lb = (bv == iota_b).astype(jnp.float32)
    pool = jnp.dot(Lb, h_dense[...], preferred_element_type=jnp.float32)
    cnt = jnp.sum(Lb, axis=1, keepdims=True)
    mean = pool / jnp.maximum(cnt, 1.0)

    @pl.when(prot == 0)
    def _():
        out[0] = mean

    @pl.when(prot == 1)
    def _():
        out[0] = out[0] + mean


def _ln(x, s, b):
    m = jnp.mean(x, -1, keepdims=True)
    v = jnp.mean(jnp.square(x - m), -1, keepdims=True)
    return (x - m) * lax.rsqrt(v + 1e-12) * s + b


def _tr_kernel(raw2, pooled2, roler, posr, hopr, table, embs, embb,
               Wq, bq, Wk, bk, Wv, bv_, Wo, bo, ln1s, ln1b,
               W1, b1, W2, b2, ln2s, ln2b, out):
    l = pl.program_id(1)
    BK = B * K // 2

    @pl.when(l == 0)
    def _():
        iota_e = lax.broadcasted_iota(jnp.int32, (128, BK), 0)
        hot = ((iota_e == roler[0]).astype(jnp.float32)
               + (iota_e == posr[0]).astype(jnp.float32)
               + (iota_e == hopr[0]).astype(jnp.float32))
        emb = lax.dot_general(
            hot, table[...], (((0,), (0,)), ((), ())),
            preferred_element_type=jnp.float32)
        h0 = raw2[...] + pooled2[...] + emb
        out[...] = _ln(h0, embs[...], embb[...])

    h = out[...]
    hb = h.astype(jnp.bfloat16)
    q = jnp.dot(hb, Wq[0], preferred_element_type=jnp.float32) + bq[0]
    kk = jnp.dot(hb, Wk[0], preferred_element_type=jnp.float32) + bk[0]
    v = jnp.dot(hb, Wv[0], preferred_element_type=jnp.float32) + bv_[0]
    nb = BK // K
    q3 = q.reshape(nb, K, D)
    k3 = kk.reshape(nb, K, D)
    v3 = v.reshape(nb, K, D)
    ctxs = []
    for hh in range(H):
        sl = slice(hh * DH, (hh + 1) * DH)
        qh, kh, vh = q3[:, :, sl], k3[:, :, sl], v3[:, :, sl]
        sc = lax.dot_general(
            qh, kh, (((2,), (2,)), ((0,), (0,))),
            preferred_element_type=jnp.float32) * _SCALE
        sc = sc - jnp.max(sc, axis=-1, keepdims=True)
        pp = jnp.exp(sc)
        pp = pp / jnp.sum(pp, axis=-1, keepdims=True)
        ctxs.append(lax.dot_general(
            pp, vh, (((2,), (1,)), ((0,), (0,))),
            preferred_element_type=jnp.float32))
    ctx = jnp.concatenate(ctxs, axis=2).reshape(BK, D)
    proj = jnp.dot(ctx.astype(jnp.bfloat16), Wo[0],
                   preferred_element_type=jnp.float32) + bo[0]
    h = _ln(h + proj, ln1s[0], ln1b[0])
    f = jax.nn.gelu(jnp.dot(h.astype(jnp.bfloat16), W1[0],
                            preferred_element_type=jnp.float32) + b1[0])
    f = jnp.dot(f.astype(jnp.bfloat16), W2[0],
                preferred_element_type=jnp.float32) + b2[0]
    out[...] = _ln(h + f, ln2s[0], ln2b[0])


def kernel(raw_features, x0, edge0, batch0, x1, edge1, batch1, role_ids,
           position_ids, hop_ids, amino_embed, gnn_W, gnn_b, role_emb,
           pos_emb, hop_emb, emb_ln_scale, emb_ln_bias, Wq, bq, Wk, bk,
           Wv, bv, Wo, bo, ln1_s, ln1_b, W1, b1, W2, b2, ln2_s, ln2_b):
    f32 = jnp.float32
    xs = jnp.concatenate([x0, x1]).astype(jnp.int32).reshape(2 * K, 1, N)
    bs = jnp.concatenate([batch0, batch1]).astype(jnp.int32).reshape(2 * K, 1, N)
    edges = jnp.concatenate([edge0, edge1]).astype(jnp.int32)
    bundle, dst_local = jax.vmap(_bucket_edges)(edges[:, 0], edges[:, 1])
    dst4 = dst_local.reshape(2 * K, NCH, 1, C)
    amino_p = jnp.zeros((128, D), f32).at[:VOCAB].set(amino_embed.astype(f32))
    b_pad = jnp.zeros((8, D), f32).at[:L_GNN].set(gnn_b.astype(f32))

    inst = lambda k, p: p * K + k
    gnn_out = pl.pallas_call(
        _gnn_kernel,
        grid=(K, 2),
        in_specs=[
            pl.BlockSpec((1, 1, N), lambda k, p: (inst(k, p), 0, 0)),
            pl.BlockSpec((1, 1, N), lambda k, p: (inst(k, p), 0, 0)),
            pl.BlockSpec((1, NCH, 1, C), lambda k, p: (inst(k, p), 0, 0, 0)),
            pl.BlockSpec((1, 1, LEN), lambda k, p: (inst(k, p), 0, 0)),
            pl.BlockSpec((128, D), lambda k, p: (0, 0)),
            pl.BlockSpec((L_GNN, D, D), lambda k, p: (0, 0, 0)),
            pl.BlockSpec((8, D), lambda k, p: (0, 0)),
        ],
        out_specs=pl.BlockSpec((1, B, D), lambda k, p: (k, 0, 0)),
        out_shape=jax.ShapeDtypeStruct((K, B, D), f32),
        scratch_shapes=[
            pltpu.VMEM((N * 8, 128), f32),
            pltpu.VMEM((N, D), f32),
            pltpu.VMEM((N, D), f32),
            pltpu.VMEM((C + 7 * S + 1, 128), f32),
            pltpu.SMEM((LEN,), jnp.int32),
            pltpu.SemaphoreType.DMA,
        ],
        compiler_params=_CP(
            dimension_semantics=("parallel", "arbitrary"),
            vmem_limit_bytes=58 * 1024 * 1024,
        ),
    )(xs, bs, dst4, bundle.reshape(2 * K, 1, LEN), amino_p, gnn_W.astype(f32), b_pad)

    pooled2 = gnn_out.transpose(1, 0, 2).reshape(B * K, D)
    raw2 = raw_features.astype(f32).reshape(B * K, D)
    roler = role_ids.astype(jnp.int32).reshape(1, B * K)
    posr = (position_ids + N_ROLES).astype(jnp.int32).reshape(1, B * K)
    hopr = (hop_ids + N_ROLES + MAX_POS).astype(jnp.int32).reshape(1, B * K)
    table = jnp.zeros((128, D), f32)
    table = table.at[:N_ROLES].set(role_emb.astype(f32))
    table = table.at[N_ROLES:N_ROLES + MAX_POS].set(pos_emb.astype(f32))
    table = table.at[N_ROLES + MAX_POS:N_ROLES + MAX_POS + N_HOPS].set(
        hop_emb.astype(f32))
    bf = jnp.bfloat16
    row = lambda a: a.astype(f32).reshape(1, -1)

    half = pl.BlockSpec((B * K // 2, D), lambda i, l: (i, 0))
    idspec = pl.BlockSpec((1, B * K // 2), lambda i, l: (0, i))
    const2 = lambda shape: pl.BlockSpec(shape, lambda i, l: (0, 0))
    lay2 = lambda d1: pl.BlockSpec((1, 1, d1), lambda i, l: (l, 0, 0))
    lay3 = lambda d1, d2: pl.BlockSpec((1, d1, d2), lambda i, l: (l, 0, 0))

    h_out = pl.pallas_call(
        _tr_kernel,
        grid=(2, L_TR),
        in_specs=[
            half, half, idspec, idspec, idspec,
            const2((128, D)), const2((1, D)), const2((1, D)),
            lay3(D, D), lay2(D), lay3(D, D), lay2(D),
            lay3(D, D), lay2(D), lay3(D, D), lay2(D),
            lay2(D), lay2(D),
            lay3(D, FF), lay2(FF), lay3(FF, D), lay2(D),
            lay2(D), lay2(D),
        ],
        out_specs=half,
        out_shape=jax.ShapeDtypeStruct((B * K, D), f32),
        compiler_params=_CP(
            dimension_semantics=("parallel", "arbitrary"),
            vmem_limit_bytes=58 * 1024 * 1024,
        ),
    )(raw2, pooled2, roler, posr, hopr, table, row(emb_ln_scale),
      row(emb_ln_bias),
      Wq.astype(bf), bq.astype(f32).reshape(L_TR, 1, D),
      Wk.astype(bf), bk.astype(f32).reshape(L_TR, 1, D),
      Wv.astype(bf), bv.astype(f32).reshape(L_TR, 1, D),
      Wo.astype(bf), bo.astype(f32).reshape(L_TR, 1, D),
      ln1_s.astype(f32).reshape(L_TR, 1, D), ln1_b.astype(f32).reshape(L_TR, 1, D),
      W1.astype(bf), b1.astype(f32).reshape(L_TR, 1, FF), W2.astype(bf), b2.astype(f32).reshape(L_TR, 1, D),
      ln2_s.astype(f32).reshape(L_TR, 1, D), ln2_b.astype(f32).reshape(L_TR, 1, D))
    return h_out.reshape(B, K, D)


# speedup vs baseline: 8.8188x; 1.0121x over previous
import functools

import jax
import jax.numpy as jnp
from jax import lax
from jax.experimental import pallas as pl
from jax.experimental.pallas import tpu as pltpu

B, K, N, E, D = 32, 32, 2048, 16384, 768
L_GNN, L_TR, H = 2, 4, 12
FF, VOCAB, N_ROLES, MAX_POS, N_HOPS = 3072, 25, 2, 64, 16

T = 256
NTILE = N // T
C = 256
S = C + 1
TOT = E + NTILE * C
NCH = TOT // C
LEN = ((NCH + TOT + 1023) // 1024) * 1024
DH = D // H
_SCALE = float(1.0 / (DH ** 0.5))

_CP = getattr(pltpu, "CompilerParams", None) or getattr(pltpu, "TPUCompilerParams")


def _bucket_edges(src, dst):
    te = dst // T
    ste, spk = lax.sort((te, dst * N + src), num_keys=1, is_stable=True)
    ssrc = spk % N
    sdst = spk // N
    tids = jnp.arange(NTILE, dtype=ste.dtype)
    counts = jnp.sum(ste[None, :] == tids[:, None], axis=1)
    padded = ((counts + C - 1) // C) * C
    cpad = jnp.cumsum(padded)
    zero = jnp.zeros((1,), counts.dtype)
    pad_off = jnp.concatenate([zero, cpad[:-1]])
    start = jnp.concatenate([zero, jnp.cumsum(counts)[:-1]])
    shift = pad_off - start
    ztail = jnp.zeros((TOT - E,), jnp.int32)
    srcp = jnp.zeros((TOT,), jnp.int32)
    dstp = jnp.zeros((TOT,), jnp.int32)
    vld = jnp.zeros((TOT,), jnp.int32)
    for t in range(NTILE):
        m = ste == t
        sv = jnp.concatenate([jnp.where(m, ssrc, 0), ztail])
        dv = jnp.concatenate([jnp.where(m, sdst, 0), ztail])
        mv = jnp.concatenate([m.astype(jnp.int32), ztail])
        srcp = srcp + jnp.roll(sv, shift[t])
        dstp = dstp + jnp.roll(dv, shift[t])
        vld = vld + jnp.roll(mv, shift[t])
    dstp = jnp.where(vld > 0, dstp, -1)
    c = jnp.arange(NCH)
    tile_c = jnp.clip(jnp.searchsorted(cpad, c * C, side="right"), 0, NTILE - 1)
    tid_off = (tile_c * T).astype(jnp.int32)
    dst_local = dstp.reshape(NCH, C) - tid_off[:, None]
    bundle = jnp.concatenate(
        [tid_off, srcp * 8, jnp.zeros((LEN - NCH - TOT,), jnp.int32)])
    return bundle, dst_local


def _gnn_kernel(xrow, batchrow, dst4, idxb, amino, Wref, bref, out,
                h_slab, h_dense, agg, tile, smem, sem):
    prot = pl.program_id(1)
    cp = pltpu.make_async_copy(idxb.at[0, 0], smem, sem)
    cp.start()
    xv = xrow[0]
    iota_v = lax.broadcasted_iota(jnp.int32, (128, N), 0)
    G = (iota_v == xv).astype(jnp.float32)
    h_dense[...] = lax.dot_general(
        G, amino[...], (((0,), (0,)), ((), ())),
        preferred_element_type=jnp.float32)
    cp.wait()

    for l in range(L_GNN):
        for j in range(D // 128):
            h_slab[j:N * 8:8, :] = h_dense[:, j * 128:(j + 1) * 128]
        agg[...] = jnp.zeros((N, D), jnp.float32)

        def chunk_body(c, _):
            base = NCH + c * C
            for mi in range(C):
                s8 = pl.multiple_of(smem[base + mi], 8)
                slab = h_slab[pl.ds(s8, 8), :]
                tile[mi:mi + 8 * S:S, :] = slab
            dstv = dst4[0, c, 0]
            iota_t = lax.broadcasted_iota(jnp.int32, (T, C), 0)
            L = (dstv[None, :] == iota_t).astype(jnp.float32)
            Hs = jnp.concatenate(
                [tile[pl.ds(j * S, C), :] for j in range(D // 128)], axis=1)
            upd = jnp.dot(L, Hs, preferred_element_type=jnp.float32)
            toff = pl.multiple_of(smem[c], T)
            agg[pl.ds(toff, T), :] = agg[pl.ds(toff, T), :] + upd
            return 0

        lax.fori_loop(0, NCH, chunk_body, 0)
        z = h_dense[...] + agg[...]
        h_dense[...] = jnp.maximum(
            jnp.dot(z, Wref[l], preferred_element_type=jnp.float32)
            + bref[l], 0.0)

    bv = batchrow[0]
    iota_b = lax.broadcasted_iota(jnp.int32, (B, N), 0)
    Lb = (bv == iota_b).astype(jnp.float32)
    pool = jnp.dot(Lb, h_dense[...], preferred_element_type=jnp.float32)
    cnt = jnp.sum(Lb, axis=1, keepdims=True)
    mean = pool / jnp.maximum(cnt, 1.0)

    @pl.when(prot == 0)
    def _():
        out[0] = mean

    @pl.when(prot == 1)
    def _():
        out[0] = out[0] + mean


def _ln(x, s, b):
    m = jnp.mean(x, -1, keepdims=True)
    v = jnp.mean(jnp.square(x - m), -1, keepdims=True)
    return (x - m) * lax.rsqrt(v + 1e-12) * s + b


def _tr_kernel(raw2, pooled2, roler, posr, hopr, table, embs, embb,
               Wq, bq, Wk, bk, Wv, bv_, Wo, bo, ln1s, ln1b,
               W1, b1, W2, b2, ln2s, ln2b, out):
    l = pl.program_id(1)
    BK = B * K // 2

    @pl.when(l == 0)
    def _():
        iota_e = lax.broadcasted_iota(jnp.int32, (128, BK), 0)
        hot = ((iota_e == roler[0]).astype(jnp.float32)
               + (iota_e == posr[0]).astype(jnp.float32)
               + (iota_e == hopr[0]).astype(jnp.float32))
        emb = lax.dot_general(
            hot, table[...], (((0,), (0,)), ((), ())),
            preferred_element_type=jnp.float32)
        h0 = raw2[...] + pooled2[...] + emb
        out[...] = _ln(h0, embs[...], embb[...])

    h = out[...]
    hb = h.astype(jnp.bfloat16)
    q = jnp.dot(hb, Wq[0], preferred_element_type=jnp.float32) + bq[0]
    kk = jnp.dot(hb, Wk[0], preferred_element_type=jnp.float32) + bk[0]
    v = jnp.dot(hb, Wv[0], preferred_element_type=jnp.float32) + bv_[0]
    nb = BK // K
    q3 = q.reshape(nb, K, D)
    k3 = kk.reshape(nb, K, D)
    v3 = v.reshape(nb, K, D)
    ctxs = []
    for hh in range(H):
        sl = slice(hh * DH, (hh + 1) * DH)
        qh, kh, vh = q3[:, :, sl], k3[:, :, sl], v3[:, :, sl]
        sc = lax.dot_general(
            qh, kh, (((2,), (2,)), ((0,), (0,))),
            preferred_element_type=jnp.float32) * _SCALE
        sc = sc - jnp.max(sc, axis=-1, keepdims=True)
        pp = jnp.exp(sc)
        pp = pp / jnp.sum(pp, axis=-1, keepdims=True)
        ctxs.append(lax.dot_general(
            pp, vh, (((2,), (1,)), ((0,), (0,))),
            preferred_element_type=jnp.float32))
    ctx = jnp.concatenate(ctxs, axis=2).reshape(BK, D)
    proj = jnp.dot(ctx.astype(jnp.bfloat16), Wo[0],
                   preferred_element_type=jnp.float32) + bo[0]
    h = _ln(h + proj, ln1s[0], ln1b[0])
    f = jax.nn.gelu(jnp.dot(h.astype(jnp.bfloat16), W1[0],
                            preferred_element_type=jnp.float32) + b1[0])
    f = jnp.dot(f.astype(jnp.bfloat16), W2[0],
                preferred_element_type=jnp.float32) + b2[0]
    out[...] = _ln(h + f, ln2s[0], ln2b[0])


def kernel(raw_features, x0, edge0, batch0, x1, edge1, batch1, role_ids,
           position_ids, hop_ids, amino_embed, gnn_W, gnn_b, role_emb,
           pos_emb, hop_emb, emb_ln_scale, emb_ln_bias, Wq, bq, Wk, bk,
           Wv, bv, Wo, bo, ln1_s, ln1_b, W1, b1, W2, b2, ln2_s, ln2_b):
    f32 = jnp.float32
    xs = jnp.concatenate([x0, x1]).astype(jnp.int32).reshape(2 * K, 1, N)
    bs = jnp.concatenate([batch0, batch1]).astype(jnp.int32).reshape(2 * K, 1, N)
    edges = jnp.concatenate([edge0, edge1]).astype(jnp.int32)
    bundle, dst_local = jax.vmap(_bucket_edges)(edges[:, 0], edges[:, 1])
    dst4 = dst_local.reshape(2 * K, NCH, 1, C)
    amino_p = jnp.zeros((128, D), f32).at[:VOCAB].set(amino_embed.astype(f32))
    b_pad = jnp.zeros((8, D), f32).at[:L_GNN].set(gnn_b.astype(f32))

    inst = lambda k, p: p * K + k
    gnn_out = pl.pallas_call(
        _gnn_kernel,
        grid=(K, 2),
        in_specs=[
            pl.BlockSpec((1, 1, N), lambda k, p: (inst(k, p), 0, 0)),
            pl.BlockSpec((1, 1, N), lambda k, p: (inst(k, p), 0, 0)),
            pl.BlockSpec((1, NCH, 1, C), lambda k, p: (inst(k, p), 0, 0, 0)),
            pl.BlockSpec((1, 1, LEN), lambda k, p: (inst(k, p), 0, 0)),
            pl.BlockSpec((128, D), lambda k, p: (0, 0)),
            pl.BlockSpec((L_GNN, D, D), lambda k, p: (0, 0, 0)),
            pl.BlockSpec((8, D), lambda k, p: (0, 0)),
        ],
        out_specs=pl.BlockSpec((1, B, D), lambda k, p: (k, 0, 0)),
        out_shape=jax.ShapeDtypeStruct((K, B, D), f32),
        scratch_shapes=[
            pltpu.VMEM((N * 8, 128), f32),
            pltpu.VMEM((N, D), f32),
            pltpu.VMEM((N, D), f32),
            pltpu.VMEM((C + 7 * S + 1, 128), f32),
            pltpu.SMEM((LEN,), jnp.int32),
            pltpu.SemaphoreType.DMA,
        ],
        compiler_params=_CP(
            dimension_semantics=("parallel", "arbitrary"),
            vmem_limit_bytes=58 * 1024 * 1024,
        ),
    )(xs, bs, dst4, bundle.reshape(2 * K, 1, LEN), amino_p, gnn_W.astype(f32), b_pad)

    pooled2 = gnn_out.transpose(1, 0, 2).reshape(B * K, D)
    raw2 = raw_features.astype(f32).reshape(B * K, D)
    roler = role_ids.astype(jnp.int32).reshape(1, B * K)
    posr = (position_ids + N_ROLES).astype(jnp.int32).reshape(1, B * K)
    hopr = (hop_ids + N_ROLES + MAX_POS).astype(jnp.int32).reshape(1, B * K)
    table = jnp.zeros((128, D), f32)
    table = table.at[:N_ROLES].set(role_emb.astype(f32))
    table = table.at[N_ROLES:N_ROLES + MAX_POS].set(pos_emb.astype(f32))
    table = table.at[N_ROLES + MAX_POS:N_ROLES + MAX_POS + N_HOPS].set(
        hop_emb.astype(f32))
    bf = jnp.bfloat16
    row = lambda a: a.astype(f32).reshape(1, -1)

    half = pl.BlockSpec((B * K // 2, D), lambda i, l: (i, 0))
    idspec = pl.BlockSpec((1, B * K // 2), lambda i, l: (0, i))
    const2 = lambda shape: pl.BlockSpec(shape, lambda i, l: (0, 0))
    lay2 = lambda d1: pl.BlockSpec((1, 1, d1), lambda i, l: (l, 0, 0))
    lay3 = lambda d1, d2: pl.BlockSpec((1, d1, d2), lambda i, l: (l, 0, 0))

    h_out = pl.pallas_call(
        _tr_kernel,
        grid=(2, L_TR),
        in_specs=[
            half, half, idspec, idspec, idspec,
            const2((128, D)), const2((1, D)), const2((1, D)),
            lay3(D, D), lay2(D), lay3(D, D), lay2(D),
            lay3(D, D), lay2(D), lay3(D, D), lay2(D),
            lay2(D), lay2(D),
            lay3(D, FF), lay2(FF), lay3(FF, D), lay2(D),
            lay2(D), lay2(D),
        ],
        out_specs=half,
        out_shape=jax.ShapeDtypeStruct((B * K, D), f32),
        compiler_params=_CP(
            dimension_semantics=("parallel", "arbitrary"),
            vmem_limit_bytes=58 * 1024 * 1024,
        ),
    )(raw2, pooled2, roler, posr, hopr, table, row(emb_ln_scale),
      row(emb_ln_bias),
      Wq.astype(bf), bq.astype(f32).reshape(L_TR, 1, D),
      Wk.astype(bf), bk.astype(f32).reshape(L_TR, 1, D),
      Wv.astype(bf), bv.astype(f32).reshape(L_TR, 1, D),
      Wo.astype(bf), bo.astype(f32).reshape(L_TR, 1, D),
      ln1_s.astype(f32).reshape(L_TR, 1, D), ln1_b.astype(f32).reshape(L_TR, 1, D),
      W1.astype(bf), b1.astype(f32).reshape(L_TR, 1, FF), W2.astype(bf), b2.astype(f32).reshape(L_TR, 1, D),
      ln2_s.astype(f32).reshape(L_TR, 1, D), ln2_b.astype(f32).reshape(L_TR, 1, D))
    return h_out.reshape(B, K, D)
